# baseline (speedup 1.0000x reference)
import functools

import jax
import jax.numpy as jnp
from jax import lax
from jax.experimental import pallas as pl
from jax.experimental.pallas import tpu as pltpu
from jax.experimental.pallas import tpu_sc as plsc

_B, _N, _M, _HA = 16, 1000, 32, 16
_NF = 17
_NM = _N * _M
_R1 = _B * _NM
_R2 = _B * _N
_TN = 200
_NT = _N // _TN
_RT = _TN * _M

_HIGH = jax.lax.Precision.DEFAULT


def _dot(a, b):
    return jnp.dot(a, b, precision=_HIGH,
                   preferred_element_type=jnp.float32)


def _softplus(x):
    return jnp.maximum(x, 0.0) + jnp.log(1.0 + jnp.exp(-jnp.abs(x)))


def _sigmoid(x):
    return 0.5 * jnp.tanh(0.5 * x) + 0.5


_SC_NC, _SC_NS = 2, 16
_SC_NW = _SC_NC * _SC_NS


def _sc_gather(table, idx):
    rows = idx.shape[0]
    per_w = rows // _SC_NW
    chunk = 4000
    n_ch = per_w // chunk
    mesh = plsc.VectorSubcoreMesh(core_axis_name="c", subcore_axis_name="s")

    @functools.partial(
        pl.kernel,
        mesh=mesh,
        compiler_params=pltpu.CompilerParams(use_tc_tiling_on_sc=False),
        out_type=jax.ShapeDtypeStruct((rows, _HA), jnp.float32),
        scratch_types=[
            pltpu.VMEM((chunk,), jnp.int32),
            pltpu.VMEM((chunk, _HA), jnp.float32),
            pltpu.SemaphoreType.DMA,
        ],
    )
    def gather_kernel(table_hbm, idx_hbm, out_hbm, idx_v, rows_v, sem):
        wid = lax.axis_index("s") * _SC_NC + lax.axis_index("c")

        @pl.loop(0, n_ch)
        def _(c):
            base = wid * per_w + c * chunk
            pltpu.sync_copy(idx_hbm.at[pl.ds(base, chunk)], idx_v)
            pltpu.async_copy(table_hbm.at[idx_v], rows_v, sem).wait()
            pltpu.sync_copy(rows_v, out_hbm.at[pl.ds(base, chunk)])

    return gather_kernel(table, idx)


def _store_x(xs, d, g, a):
    flt = lax.broadcasted_iota(jnp.int32, (1, _NF), 1).astype(jnp.float32) * 0.5
    d17 = _dot(d, jnp.ones((1, _NF), jnp.float32))
    gauss = jnp.exp((d17 - flt) ** 2 * -4.0)
    arep = jnp.broadcast_to(a[:, None, :], (_TN, _M, _HA)).reshape(_RT, _HA)
    xs[:, 0:_HA] = arep
    xs[:, _HA:2 * _HA] = g
    xs[:, 2 * _HA:49] = gauss


def _layer_body(refs, *, atom_3d, head_refs):
    (d_ref, g_ref, a_ref, wf_ref, wc_ref,
     p_ref, out_ref, acc, summed, xs) = refs
    p = pl.program_id(0)
    b = pl.program_id(1)
    t = pl.program_id(2)
    prm = p_ref[...]
    bff, bfc = prm[0:1, :], prm[1:2, :]
    g1f, g1c = prm[2:3, :], prm[3:4, :]
    b1f, b1c = prm[4:5, :], prm[5:6, :]
    g2, b2 = prm[6:7, :], prm[7:8, :]
    a = a_ref[0] if atom_3d else a_ref[...]
    off = pl.multiple_of(b * _N + t * _TN, _TN)

    @pl.when((p == 0) & (b == 0) & (t == 0))
    def _():
        acc[...] = jnp.zeros((72, 128), jnp.float32)
        xs[:, 49:50] = jnp.ones((_RT, 1), jnp.float32)
        xs[:, 50:64] = jnp.zeros((_RT, 14), jnp.float32)

    @pl.when(p == 0)
    def _():
        _store_x(xs, d_ref[0], g_ref[0], a)
        x = xs[...]
        s = lax.dot_general(x, x, (((0,), (0,)), ((), ())),
                            preferred_element_type=jnp.float32)
        acc[0:64, 0:64] += s

    def _bn1_fold(w, bias1, gamma1, beta1):
        sm = acc[0:64, 0:64]
        tq = _dot(sm, w)
        q = jnp.sum(w * tq, axis=0, keepdims=True) * (1.0 / _R1)
        mu0 = tq[49:50, :] * (1.0 / _R1)
        var = q - mu0 * mu0
        alpha = gamma1 * lax.rsqrt(var + 1e-5)
        return w * alpha, beta1 - mu0 * alpha

    @pl.when(p == 1)
    def _():
        _store_x(xs, d_ref[0], g_ref[0], a)
        x = xs[...]
        wbf, cbf = _bn1_fold(wf_ref[...], bff, g1f, b1f)
        wbc, cbc = _bn1_fold(wc_ref[...], bfc, g1c, b1c)
        xf = _dot(x, wbf) + cbf
        xc = _dot(x, wbc) + cbc
        act = _sigmoid(xf) * _softplus(xc)
        sm = act.reshape(_TN, _M, _HA).sum(axis=1)
        acc[65:66, 0:16] += jnp.sum(sm, axis=0, keepdims=True)
        acc[66:67, 0:16] += jnp.sum(sm * sm, axis=0, keepdims=True)
        summed[pl.ds(off, _TN), :] = sm

    @pl.when(p == 2)
    def _():
        mu2 = acc[65:66, 0:16] * (1.0 / _R2)
        var2 = acc[66:67, 0:16] * (1.0 / _R2) - mu2 * mu2
        sm = summed[pl.ds(off, _TN), :]
        upd = a + _softplus(a + (sm - mu2) * lax.rsqrt(var2 + 1e-5) * g2 + b2)
        if head_refs is None:
            out_ref[0] = upd
        else:
            aw_ref, ab_ref, fw_ref, fb_ref = head_refs
            r = jnp.maximum(upd, 0.0)

            @pl.when(t == 0)
            def _():
                acc[67:68, 0:16] = jnp.zeros((1, 16), jnp.float32)

            acc[67:68, 0:16] += jnp.sum(r, axis=0, keepdims=True)

            @pl.when(t == _NT - 1)
            def _():
                e = acc[67:68, 0:16] * (1.0 / _N)
                h = _dot(e, aw_ref[...]) + ab_ref[...]
                lg = _dot(h, fw_ref[...]) + fb_ref[...]
                ex = jnp.exp(lg - jnp.max(lg, axis=-1, keepdims=True))
                out_ref[0] = ex / jnp.sum(ex, axis=-1, keepdims=True)


def _split_weights(Wf, bf, g1, b1, g2, b2):
    wfull = jnp.concatenate(
        [Wf, jnp.zeros((64 - 49, 2 * _HA), jnp.float32)], axis=0)
    wf, wc = wfull[:, 0:_HA], wfull[:, _HA:]
    prm = jnp.stack([bf[:_HA], bf[_HA:], g1[:_HA], g1[_HA:],
                     b1[:_HA], b1[_HA:], g2, b2], axis=0)
    return wf, wc, prm


def _full_spec(shape):
    return pl.BlockSpec(shape, lambda p, b, t: (0,) * len(shape))


def _rows_spec():
    def imap(p, b, t):
        live = p < 2
        return (jnp.where(live, b, 0), jnp.where(live, t, 0), 0)
    return imap


def _conv_layer1(d_rows, gath, emb_table, wargs):
    wf, wc, prm = wargs

    def body(*refs):
        _layer_body(refs, atom_3d=False, head_refs=None)

    grid = (3, _B, _NT)
    return pl.pallas_call(
        body,
        grid=grid,
        in_specs=[
            pl.BlockSpec((1, _RT, 1), _rows_spec()),
            pl.BlockSpec((1, _RT, _HA), _rows_spec()),
            pl.BlockSpec((_TN, _HA), lambda p, b, t: (t, 0)),
            _full_spec((64, _HA)), _full_spec((64, _HA)),
            _full_spec((8, _HA)),
        ],
        out_specs=pl.BlockSpec(
            (1, _TN, _HA),
            lambda p, b, t: (jnp.where(p == 2, b, 0),
                             jnp.where(p == 2, t, 0), 0)),
        out_shape=jax.ShapeDtypeStruct((_B, _N, _HA), jnp.float32),
        scratch_shapes=[
            pltpu.VMEM((72, 128), jnp.float32),
            pltpu.VMEM((_R2, _HA), jnp.float32),
            pltpu.VMEM((_RT, 64), jnp.float32),
        ],
    )(d_rows, gath, emb_table, wf, wc, prm)


def _conv_layer2_head(d_rows, gath, atom1, wargs, amino_W, amino_b, fc_W, fc_b):
    wf, wc, prm = wargs

    def body_wrap(d_ref, g_ref, a_ref, r3, r4, p_ref,
                  aw_ref, ab_ref, fw_ref, fb_ref, out_ref, acc, summed, xs):
        _layer_body(
            (d_ref, g_ref, a_ref, r3, r4, p_ref, out_ref,
             acc, summed, xs),
            atom_3d=True, head_refs=(aw_ref, ab_ref, fw_ref, fb_ref))

    grid = (3, _B, _NT)
    return pl.pallas_call(
        body_wrap,
        grid=grid,
        in_specs=[
            pl.BlockSpec((1, _RT, 1), _rows_spec()),
            pl.BlockSpec((1, _RT, _HA), _rows_spec()),
            pl.BlockSpec((1, _TN, _HA), lambda p, b, t: (b, t, 0)),
            _full_spec((64, _HA)), _full_spec((64, _HA)),
            _full_spec((8, _HA)),
            _full_spec((_HA, 32)), _full_spec((1, 32)),
            _full_spec((32, 8)), _full_spec((1, 8)),
        ],
        out_specs=pl.BlockSpec(
            (1, 1, 8), lambda p, b, t: (jnp.where(p == 2, b, 0), 0, 0)),
        out_shape=jax.ShapeDtypeStruct((_B, 1, 8), jnp.float32),
        scratch_shapes=[
            pltpu.VMEM((72, 128), jnp.float32),
            pltpu.VMEM((_R2, _HA), jnp.float32),
            pltpu.VMEM((_RT, 64), jnp.float32),
        ],
    )(d_rows, gath, atom1, wf, wc, prm,
      amino_W, amino_b.reshape(1, 32), fc_W, fc_b.reshape(1, 8))


def kernel(data, emb_table, Wf0, bf0, g10, b10, g20, b20, Wf1, bf1, g11, b11,
           g21, b21, amino_W, amino_b, fc_W, fc_b):
    dist = data[:, :, :_M]
    idx = data[:, :, _M:].astype(jnp.int32)
    d_rows = dist.reshape(_B, _NM, 1)
    idx1 = idx.reshape(_R1)
    idx2 = (idx + (jnp.arange(_B, dtype=jnp.int32) * _N)[:, None, None]
            ).reshape(_R1)

    w0 = _split_weights(Wf0, bf0, g10, b10, g20, b20)
    w1 = _split_weights(Wf1, bf1, g11, b11, g21, b21)

    gath1 = _sc_gather(emb_table, idx1).reshape(_B, _NM, _HA)
    atom1 = _conv_layer1(d_rows, gath1, emb_table, w0)
    gath2 = _sc_gather(atom1.reshape(_R2, _HA), idx2).reshape(_B, _NM, _HA)
    probs = _conv_layer2_head(d_rows, gath2, atom1, w1,
                              amino_W, amino_b, fc_W, fc_b)
    return probs.reshape(_B, 8)


# speedup vs baseline: 1.0263x; 1.0263x over previous
import functools

import jax
import jax.numpy as jnp
from jax import lax
from jax.experimental import pallas as pl
from jax.experimental.pallas import tpu as pltpu
from jax.experimental.pallas import tpu_sc as plsc

_B, _N, _M, _HA = 16, 1000, 32, 16
_NF = 17
_NM = _N * _M
_R1 = _B * _NM
_R2 = _B * _N
_TN = 200
_NT = _N // _TN
_RT = _TN * _M

_HIGH = jax.lax.Precision.DEFAULT


def _dot(a, b):
    return jnp.dot(a, b, precision=_HIGH,
                   preferred_element_type=jnp.float32)


def _softplus(x):
    return jnp.maximum(x, 0.0) + jnp.log(1.0 + jnp.exp(-jnp.abs(x)))


def _sigmoid(x):
    return 0.5 * jnp.tanh(0.5 * x) + 0.5


_SC_NC, _SC_NS = 2, 16
_SC_NW = _SC_NC * _SC_NS


def _sc_gather(table, idx):
    rows = idx.shape[0]
    per_w = rows // _SC_NW
    chunk = 4000
    n_ch = per_w // chunk
    mesh = plsc.VectorSubcoreMesh(core_axis_name="c", subcore_axis_name="s")

    @functools.partial(
        pl.kernel,
        mesh=mesh,
        compiler_params=pltpu.CompilerParams(use_tc_tiling_on_sc=False),
        out_type=jax.ShapeDtypeStruct((rows, _HA), jnp.float32),
        scratch_types=[
            pltpu.VMEM((chunk,), jnp.int32),
            pltpu.VMEM((chunk, _HA), jnp.float32),
            pltpu.SemaphoreType.DMA,
        ],
    )
    def gather_kernel(table_hbm, idx_hbm, out_hbm, idx_v, rows_v, sem):
        wid = lax.axis_index("s") * _SC_NC + lax.axis_index("c")

        @pl.loop(0, n_ch)
        def _(c):
            base = wid * per_w + c * chunk
            pltpu.sync_copy(idx_hbm.at[pl.ds(base, chunk)], idx_v)
            pltpu.async_copy(table_hbm.at[idx_v], rows_v, sem).wait()
            pltpu.sync_copy(rows_v, out_hbm.at[pl.ds(base, chunk)])

    return gather_kernel(table, idx)


def _build_x(d, g, a):
    flt = lax.broadcasted_iota(jnp.int32, (1, _NF), 1).astype(jnp.float32) * 0.5
    d17 = _dot(d, jnp.ones((1, _NF), jnp.float32))
    gauss = jnp.exp((d17 - flt) ** 2 * -4.0)
    arep = jnp.broadcast_to(a[:, None, :], (_TN, _M, _HA)).reshape(_RT, _HA)
    ones = jnp.ones((_RT, 1), jnp.float32)
    return jnp.concatenate([arep, g, gauss, ones], axis=1)


def _layer_body(refs, *, atom_3d, head_refs):
    (d_ref, g_ref, a_ref, wf_ref, wc_ref,
     p_ref, out_ref, acc, summed) = refs
    p = pl.program_id(0)
    b = pl.program_id(1)
    t = pl.program_id(2)
    prm = p_ref[...]
    bff, bfc = prm[0:1, :], prm[1:2, :]
    g1f, g1c = prm[2:3, :], prm[3:4, :]
    b1f, b1c = prm[4:5, :], prm[5:6, :]
    g2, b2 = prm[6:7, :], prm[7:8, :]
    a = a_ref[0] if atom_3d else a_ref[...]
    off = pl.multiple_of(b * _N + t * _TN, _TN)

    def x_now():
        return _build_x(d_ref[0], g_ref[0], a)

    @pl.when((p == 0) & (b == 0) & (t == 0))
    def _():
        acc[...] = jnp.zeros((56, 128), jnp.float32)

    @pl.when(p == 0)
    def _():
        x = x_now()
        s = lax.dot_general(x, x, (((0,), (0,)), ((), ())),
                            preferred_element_type=jnp.float32)
        acc[0:50, 0:50] += s

    def _bn1_fold(w, bias1, gamma1, beta1):
        sm = acc[0:50, 0:50]
        w50 = jnp.concatenate([w, jnp.zeros((1, _HA), jnp.float32)], axis=0)
        tq = _dot(sm, w50)
        q = jnp.sum(w50 * tq, axis=0, keepdims=True) * (1.0 / _R1)
        mu0 = tq[49:50, :] * (1.0 / _R1)
        var = q - mu0 * mu0
        alpha = gamma1 * lax.rsqrt(var + 1e-5)
        return w50 * alpha, beta1 - mu0 * alpha

    @pl.when(p == 1)
    def _():
        x = x_now()
        wbf, cbf = _bn1_fold(wf_ref[...], bff, g1f, b1f)
        wbc, cbc = _bn1_fold(wc_ref[...], bfc, g1c, b1c)
        xf = _dot(x, wbf) + cbf
        xc = _dot(x, wbc) + cbc
        act = _sigmoid(xf) * _softplus(xc)
        sm = act.reshape(_TN, _M, _HA).sum(axis=1)
        acc[52:53, 0:16] += jnp.sum(sm, axis=0, keepdims=True)
        acc[53:54, 0:16] += jnp.sum(sm * sm, axis=0, keepdims=True)
        summed[pl.ds(off, _TN), :] = sm

    @pl.when(p == 2)
    def _():
        mu2 = acc[52:53, 0:16] * (1.0 / _R2)
        var2 = acc[53:54, 0:16] * (1.0 / _R2) - mu2 * mu2
        sm = summed[pl.ds(off, _TN), :]
        upd = a + _softplus(a + (sm - mu2) * lax.rsqrt(var2 + 1e-5) * g2 + b2)
        if head_refs is None:
            out_ref[0] = upd
        else:
            aw_ref, ab_ref, fw_ref, fb_ref = head_refs
            r = jnp.maximum(upd, 0.0)

            @pl.when(t == 0)
            def _():
                acc[54:55, 0:16] = jnp.zeros((1, 16), jnp.float32)

            acc[54:55, 0:16] += jnp.sum(r, axis=0, keepdims=True)

            @pl.when(t == _NT - 1)
            def _():
                e = acc[54:55, 0:16] * (1.0 / _N)
                h = _dot(e, aw_ref[...]) + ab_ref[...]
                lg = _dot(h, fw_ref[...]) + fb_ref[...]
                ex = jnp.exp(lg - jnp.max(lg, axis=-1, keepdims=True))
                out_ref[0] = ex / jnp.sum(ex, axis=-1, keepdims=True)


def _split_weights(Wf, bf, g1, b1, g2, b2):
    wf, wc = Wf[:, 0:_HA], Wf[:, _HA:]
    prm = jnp.stack([bf[:_HA], bf[_HA:], g1[:_HA], g1[_HA:],
                     b1[:_HA], b1[_HA:], g2, b2], axis=0)
    return wf, wc, prm


def _full_spec(shape):
    return pl.BlockSpec(shape, lambda p, b, t: (0,) * len(shape))


def _rows_spec():
    def imap(p, b, t):
        live = p < 2
        return (jnp.where(live, b, 0), jnp.where(live, t, 0), 0)
    return imap


def _conv_layer1(d_rows, gath, emb_table, wargs):
    wf, wc, prm = wargs

    def body(*refs):
        _layer_body(refs, atom_3d=False, head_refs=None)

    grid = (3, _B, _NT)
    return pl.pallas_call(
        body,
        grid=grid,
        in_specs=[
            pl.BlockSpec((1, _RT, 1), _rows_spec()),
            pl.BlockSpec((1, _RT, _HA), _rows_spec()),
            pl.BlockSpec((_TN, _HA), lambda p, b, t: (t, 0)),
            _full_spec((2 * _HA + _NF, _HA)), _full_spec((2 * _HA + _NF, _HA)),
            _full_spec((8, _HA)),
        ],
        out_specs=pl.BlockSpec(
            (1, _TN, _HA),
            lambda p, b, t: (jnp.where(p == 2, b, 0),
                             jnp.where(p == 2, t, 0), 0)),
        out_shape=jax.ShapeDtypeStruct((_B, _N, _HA), jnp.float32),
        scratch_shapes=[
            pltpu.VMEM((56, 128), jnp.float32),
            pltpu.VMEM((_R2, _HA), jnp.float32),
        ],
    )(d_rows, gath, emb_table, wf, wc, prm)


def _conv_layer2_head(d_rows, gath, atom1, wargs, amino_W, amino_b, fc_W, fc_b):
    wf, wc, prm = wargs

    def body_wrap(d_ref, g_ref, a_ref, r3, r4, p_ref,
                  aw_ref, ab_ref, fw_ref, fb_ref, out_ref, acc, summed):
        _layer_body(
            (d_ref, g_ref, a_ref, r3, r4, p_ref, out_ref,
             acc, summed),
            atom_3d=True, head_refs=(aw_ref, ab_ref, fw_ref, fb_ref))

    grid = (3, _B, _NT)
    return pl.pallas_call(
        body_wrap,
        grid=grid,
        in_specs=[
            pl.BlockSpec((1, _RT, 1), _rows_spec()),
            pl.BlockSpec((1, _RT, _HA), _rows_spec()),
            pl.BlockSpec((1, _TN, _HA), lambda p, b, t: (b, t, 0)),
            _full_spec((2 * _HA + _NF, _HA)), _full_spec((2 * _HA + _NF, _HA)),
            _full_spec((8, _HA)),
            _full_spec((_HA, 32)), _full_spec((1, 32)),
            _full_spec((32, 8)), _full_spec((1, 8)),
        ],
        out_specs=pl.BlockSpec(
            (1, 1, 8), lambda p, b, t: (jnp.where(p == 2, b, 0), 0, 0)),
        out_shape=jax.ShapeDtypeStruct((_B, 1, 8), jnp.float32),
        scratch_shapes=[
            pltpu.VMEM((56, 128), jnp.float32),
            pltpu.VMEM((_R2, _HA), jnp.float32),
        ],
    )(d_rows, gath, atom1, wf, wc, prm,
      amino_W, amino_b.reshape(1, 32), fc_W, fc_b.reshape(1, 8))


def kernel(data, emb_table, Wf0, bf0, g10, b10, g20, b20, Wf1, bf1, g11, b11,
           g21, b21, amino_W, amino_b, fc_W, fc_b):
    dist = data[:, :, :_M]
    idx = data[:, :, _M:].astype(jnp.int32)
    d_rows = dist.reshape(_B, _NM, 1)
    idx1 = idx.reshape(_R1)
    idx2 = (idx + (jnp.arange(_B, dtype=jnp.int32) * _N)[:, None, None]
            ).reshape(_R1)

    w0 = _split_weights(Wf0, bf0, g10, b10, g20, b20)
    w1 = _split_weights(Wf1, bf1, g11, b11, g21, b21)

    gath1 = _sc_gather(emb_table, idx1).reshape(_B, _NM, _HA)
    atom1 = _conv_layer1(d_rows, gath1, emb_table, w0)
    gath2 = _sc_gather(atom1.reshape(_R2, _HA), idx2).reshape(_B, _NM, _HA)
    probs = _conv_layer2_head(d_rows, gath2, atom1, w1,
                              amino_W, amino_b, fc_W, fc_b)
    return probs.reshape(_B, 8)


# speedup vs baseline: 1.0283x; 1.0019x over previous
import functools

import jax
import jax.numpy as jnp
from jax import lax
from jax.experimental import pallas as pl
from jax.experimental.pallas import tpu as pltpu
from jax.experimental.pallas import tpu_sc as plsc

_B, _N, _M, _HA = 16, 1000, 32, 16
_NF = 17
_NM = _N * _M
_R1 = _B * _NM
_R2 = _B * _N
_TN = 200
_NT = _N // _TN
_RT = _TN * _M

_HIGH = jax.lax.Precision.DEFAULT


def _dot(a, b):
    return jnp.dot(a, b, precision=_HIGH,
                   preferred_element_type=jnp.float32)


def _softplus(x):
    return jnp.maximum(x, 0.0) + jnp.log(1.0 + jnp.exp(-jnp.abs(x)))


def _sigmoid(x):
    return 0.5 * jnp.tanh(0.5 * x) + 0.5


_SC_NC, _SC_NS = 2, 16
_SC_NW = _SC_NC * _SC_NS


def _sc_gather(table, idx):
    rows = idx.shape[0]
    per_w = rows // _SC_NW
    chunk = 2000
    n_ch = per_w // chunk
    mesh = plsc.VectorSubcoreMesh(core_axis_name="c", subcore_axis_name="s")

    @functools.partial(
        pl.kernel,
        mesh=mesh,
        compiler_params=pltpu.CompilerParams(use_tc_tiling_on_sc=False),
        out_type=jax.ShapeDtypeStruct((rows, _HA), jnp.float32),
        scratch_types=[
            pltpu.VMEM((chunk,), jnp.int32),
            pltpu.VMEM((chunk,), jnp.int32),
            pltpu.VMEM((chunk, _HA), jnp.float32),
            pltpu.VMEM((chunk, _HA), jnp.float32),
            pltpu.SemaphoreType.DMA,
            pltpu.SemaphoreType.DMA,
            pltpu.SemaphoreType.DMA,
            pltpu.SemaphoreType.DMA,
        ],
    )
    def gather_kernel(table_hbm, idx_hbm, out_hbm, idx0, idx1, rows0, rows1,
                      gsem0, gsem1, ssem0, ssem1):
        wid = lax.axis_index("s") * _SC_NC + lax.axis_index("c")
        idx_v = (idx0, idx1)
        rows_v = (rows0, rows1)
        gsem = (gsem0, gsem1)
        ssem = (ssem0, ssem1)
        gathers = [None, None]
        stores = [None, None]
        for c in range(n_ch):
            u = c % 2
            base = wid * per_w + c * chunk
            if stores[u] is not None:
                stores[u].wait()
                stores[u] = None
            pltpu.sync_copy(idx_hbm.at[pl.ds(base, chunk)], idx_v[u])
            gathers[u] = pltpu.async_copy(
                table_hbm.at[idx_v[u]], rows_v[u], gsem[u])
            if gathers[1 - u] is not None:
                gathers[1 - u].wait()
                gathers[1 - u] = None
                pbase = wid * per_w + (c - 1) * chunk
                stores[1 - u] = pltpu.async_copy(
                    rows_v[1 - u], out_hbm.at[pl.ds(pbase, chunk)],
                    ssem[1 - u])
        u_last = (n_ch - 1) % 2
        gathers[u_last].wait()
        pltpu.sync_copy(rows_v[u_last],
                        out_hbm.at[pl.ds(wid * per_w + (n_ch - 1) * chunk,
                                         chunk)])
        if stores[1 - u_last] is not None:
            stores[1 - u_last].wait()

    return gather_kernel(table, idx)


def _build_x(d, g, a):
    flt = lax.broadcasted_iota(jnp.int32, (1, _NF), 1).astype(jnp.float32) * 0.5
    d17 = _dot(d, jnp.ones((1, _NF), jnp.float32))
    gauss = jnp.exp((d17 - flt) ** 2 * -4.0)
    arep = jnp.broadcast_to(a[:, None, :], (_TN, _M, _HA)).reshape(_RT, _HA)
    ones = jnp.ones((_RT, 1), jnp.float32)
    return jnp.concatenate([arep, g, gauss, ones], axis=1)


def _layer_body(refs, *, atom_3d, head_refs):
    (d_ref, g_ref, a_ref, wf_ref, wc_ref,
     p_ref, out_ref, acc, summed) = refs
    p = pl.program_id(0)
    b = pl.program_id(1)
    t = pl.program_id(2)
    prm = p_ref[...]
    bff, bfc = prm[0:1, :], prm[1:2, :]
    g1f, g1c = prm[2:3, :], prm[3:4, :]
    b1f, b1c = prm[4:5, :], prm[5:6, :]
    g2, b2 = prm[6:7, :], prm[7:8, :]
    a = a_ref[0] if atom_3d else a_ref[...]
    off = pl.multiple_of(b * _N + t * _TN, _TN)

    def x_now():
        return _build_x(d_ref[0], g_ref[0], a)

    @pl.when((p == 0) & (b == 0) & (t == 0))
    def _():
        acc[...] = jnp.zeros((56, 128), jnp.float32)

    @pl.when(p == 0)
    def _():
        x = x_now()
        s = lax.dot_general(x, x, (((0,), (0,)), ((), ())),
                            preferred_element_type=jnp.float32)
        acc[0:50, 0:50] += s

    def _bn1_fold(w, bias1, gamma1, beta1):
        sm = acc[0:50, 0:50]
        w50 = jnp.concatenate([w, jnp.zeros((1, _HA), jnp.float32)], axis=0)
        tq = _dot(sm, w50)
        q = jnp.sum(w50 * tq, axis=0, keepdims=True) * (1.0 / _R1)
        mu0 = tq[49:50, :] * (1.0 / _R1)
        var = q - mu0 * mu0
        alpha = gamma1 * lax.rsqrt(var + 1e-5)
        return w50 * alpha, beta1 - mu0 * alpha

    @pl.when(p == 1)
    def _():
        x = x_now()
        wbf, cbf = _bn1_fold(wf_ref[...], bff, g1f, b1f)
        wbc, cbc = _bn1_fold(wc_ref[...], bfc, g1c, b1c)
        xf = _dot(x, wbf) + cbf
        xc = _dot(x, wbc) + cbc
        act = _sigmoid(xf) * _softplus(xc)
        sm = act.reshape(_TN, _M, _HA).sum(axis=1)
        acc[52:53, 0:16] += jnp.sum(sm, axis=0, keepdims=True)
        acc[53:54, 0:16] += jnp.sum(sm * sm, axis=0, keepdims=True)
        summed[pl.ds(off, _TN), :] = sm

    @pl.when(p == 2)
    def _():
        mu2 = acc[52:53, 0:16] * (1.0 / _R2)
        var2 = acc[53:54, 0:16] * (1.0 / _R2) - mu2 * mu2
        sm = summed[pl.ds(off, _TN), :]
        upd = a + _softplus(a + (sm - mu2) * lax.rsqrt(var2 + 1e-5) * g2 + b2)
        if head_refs is None:
            out_ref[0] = upd
        else:
            aw_ref, ab_ref, fw_ref, fb_ref = head_refs
            r = jnp.maximum(upd, 0.0)

            @pl.when(t == 0)
            def _():
                acc[54:55, 0:16] = jnp.zeros((1, 16), jnp.float32)

            acc[54:55, 0:16] += jnp.sum(r, axis=0, keepdims=True)

            @pl.when(t == _NT - 1)
            def _():
                e = acc[54:55, 0:16] * (1.0 / _N)
                h = _dot(e, aw_ref[...]) + ab_ref[...]
                lg = _dot(h, fw_ref[...]) + fb_ref[...]
                ex = jnp.exp(lg - jnp.max(lg, axis=-1, keepdims=True))
                out_ref[0] = ex / jnp.sum(ex, axis=-1, keepdims=True)


def _split_weights(Wf, bf, g1, b1, g2, b2):
    wf, wc = Wf[:, 0:_HA], Wf[:, _HA:]
    prm = jnp.stack([bf[:_HA], bf[_HA:], g1[:_HA], g1[_HA:],
                     b1[:_HA], b1[_HA:], g2, b2], axis=0)
    return wf, wc, prm


def _full_spec(shape):
    return pl.BlockSpec(shape, lambda p, b, t: (0,) * len(shape))


def _rows_spec():
    def imap(p, b, t):
        live = p < 2
        return (jnp.where(live, b, 0), jnp.where(live, t, 0), 0)
    return imap


def _conv_layer1(d_rows, gath, emb_table, wargs):
    wf, wc, prm = wargs

    def body(*refs):
        _layer_body(refs, atom_3d=False, head_refs=None)

    grid = (3, _B, _NT)
    return pl.pallas_call(
        body,
        grid=grid,
        in_specs=[
            pl.BlockSpec((1, _RT, 1), _rows_spec()),
            pl.BlockSpec((1, _RT, _HA), _rows_spec()),
            pl.BlockSpec((_TN, _HA), lambda p, b, t: (t, 0)),
            _full_spec((2 * _HA + _NF, _HA)), _full_spec((2 * _HA + _NF, _HA)),
            _full_spec((8, _HA)),
        ],
        out_specs=pl.BlockSpec(
            (1, _TN, _HA),
            lambda p, b, t: (jnp.where(p == 2, b, 0),
                             jnp.where(p == 2, t, 0), 0)),
        out_shape=jax.ShapeDtypeStruct((_B, _N, _HA), jnp.float32),
        scratch_shapes=[
            pltpu.VMEM((56, 128), jnp.float32),
            pltpu.VMEM((_R2, _HA), jnp.float32),
        ],
    )(d_rows, gath, emb_table, wf, wc, prm)


def _conv_layer2_head(d_rows, gath, atom1, wargs, amino_W, amino_b, fc_W, fc_b):
    wf, wc, prm = wargs

    def body_wrap(d_ref, g_ref, a_ref, r3, r4, p_ref,
                  aw_ref, ab_ref, fw_ref, fb_ref, out_ref, acc, summed):
        _layer_body(
            (d_ref, g_ref, a_ref, r3, r4, p_ref, out_ref,
             acc, summed),
            atom_3d=True, head_refs=(aw_ref, ab_ref, fw_ref, fb_ref))

    grid = (3, _B, _NT)
    return pl.pallas_call(
        body_wrap,
        grid=grid,
        in_specs=[
            pl.BlockSpec((1, _RT, 1), _rows_spec()),
            pl.BlockSpec((1, _RT, _HA), _rows_spec()),
            pl.BlockSpec((1, _TN, _HA), lambda p, b, t: (b, t, 0)),
            _full_spec((2 * _HA + _NF, _HA)), _full_spec((2 * _HA + _NF, _HA)),
            _full_spec((8, _HA)),
            _full_spec((_HA, 32)), _full_spec((1, 32)),
            _full_spec((32, 8)), _full_spec((1, 8)),
        ],
        out_specs=pl.BlockSpec(
            (1, 1, 8), lambda p, b, t: (jnp.where(p == 2, b, 0), 0, 0)),
        out_shape=jax.ShapeDtypeStruct((_B, 1, 8), jnp.float32),
        scratch_shapes=[
            pltpu.VMEM((56, 128), jnp.float32),
            pltpu.VMEM((_R2, _HA), jnp.float32),
        ],
    )(d_rows, gath, atom1, wf, wc, prm,
      amino_W, amino_b.reshape(1, 32), fc_W, fc_b.reshape(1, 8))


def kernel(data, emb_table, Wf0, bf0, g10, b10, g20, b20, Wf1, bf1, g11, b11,
           g21, b21, amino_W, amino_b, fc_W, fc_b):
    dist = data[:, :, :_M]
    idx = data[:, :, _M:].astype(jnp.int32)
    d_rows = dist.reshape(_B, _NM, 1)
    idx1 = idx.reshape(_R1)
    idx2 = (idx + (jnp.arange(_B, dtype=jnp.int32) * _N)[:, None, None]
            ).reshape(_R1)

    w0 = _split_weights(Wf0, bf0, g10, b10, g20, b20)
    w1 = _split_weights(Wf1, bf1, g11, b11, g21, b21)

    gath1 = _sc_gather(emb_table, idx1).reshape(_B, _NM, _HA)
    atom1 = _conv_layer1(d_rows, gath1, emb_table, w0)
    gath2 = _sc_gather(atom1.reshape(_R2, _HA), idx2).reshape(_B, _NM, _HA)
    probs = _conv_layer2_head(d_rows, gath2, atom1, w1,
                              amino_W, amino_b, fc_W, fc_b)
    return probs.reshape(_B, 8)
